# Initial kernel scaffold; baseline (speedup 1.0000x reference)
import jax
import jax.numpy as jnp
from jax.experimental import pallas as pl


def kernel(x, edge_index, batch, enc_W, enc_b, enc_g, enc_beta, gat_W, gat_asrc, gat_adst, gat_b, bn1_g, bn1_b, sage2_Wl, sage2_bl, sage2_Wr, bn2_g, bn2_b, sage3_Wl, sage3_bl, sage3_Wr, bn3_g, bn3_b, c1_W, c1_b, c2_W, c2_b, c3_W, c3_b):
    raise NotImplementedError("write your pallas kernel here")


# speedup vs baseline: 6.0416x; 6.0416x over previous
import math
import functools

import jax
import jax.numpy as jnp
from jax import lax
from jax.experimental import pallas as pl
from jax.experimental.pallas import tpu as pltpu

N = 10000
E = 320000
D = 128
H = 4
DH = 32
G = 64

BN_SCALE = 1.0 / math.sqrt(1.0 + 1e-5)

ROWS = 1000
GRID = N // ROWS
TW = 144


def _full(i):
    return 0


def _k1_body(x_ref, encW_ref, encb_ref, encg_ref, encbeta_ref, gatW_ref,
             abds_ref, abdd_ref, tsrc_ref, tad_ref):
    x = x_ref[...]
    h0 = jnp.dot(x, encW_ref[...], preferred_element_type=jnp.float32)
    h0 = h0 + encb_ref[...]
    h0 = encg_ref[...] * h0 * BN_SCALE + encbeta_ref[...]
    h0 = jnp.maximum(h0, 0.0)
    h1 = jnp.dot(h0, gatW_ref[...], preferred_element_type=jnp.float32)
    av = jnp.dot(h1, abds_ref[...], preferred_element_type=jnp.float32)
    ad = jnp.dot(h1, abdd_ref[...], preferred_element_type=jnp.float32)
    tsrc_ref[:, 0:D] = h1
    tsrc_ref[:, D:TW] = av
    tad_ref[...] = ad


def _run_k1(x, enc_W, enc_b, enc_g, enc_beta, gat_W, abd_s, abd_d):
    return pl.pallas_call(
        _k1_body,
        grid=(GRID,),
        in_specs=[
            pl.BlockSpec((ROWS, D), lambda i: (i, 0)),
            pl.BlockSpec((D, D), lambda i: (0, 0)),
            pl.BlockSpec((D,), _full),
            pl.BlockSpec((D,), _full),
            pl.BlockSpec((D,), _full),
            pl.BlockSpec((D, D), lambda i: (0, 0)),
            pl.BlockSpec((D, 16), lambda i: (0, 0)),
            pl.BlockSpec((D, 16), lambda i: (0, 0)),
        ],
        out_specs=[
            pl.BlockSpec((ROWS, TW), lambda i: (i, 0)),
            pl.BlockSpec((ROWS, 16), lambda i: (i, 0)),
        ],
        out_shape=[
            jax.ShapeDtypeStruct((N, TW), jnp.float32),
            jax.ShapeDtypeStruct((N, 16), jnp.float32),
        ],
    )(x, enc_W, enc_b, enc_g, enc_beta, gat_W, abd_s, abd_d)


def _k2_body(msg0_ref, msg1_ref, den0_ref, den1_ref, exp_ref, gatb_ref,
             g_ref, b_ref, t2_ref):
    msg = msg0_ref[...] + msg1_ref[...]
    den = den0_ref[...] + den1_ref[...]
    dd = jnp.dot(den, exp_ref[...], preferred_element_type=jnp.float32)
    msgn = jnp.where(dd > 0.0, msg / jnp.where(dd > 0.0, dd, 1.0), 0.0)
    msgn = msgn + gatb_ref[...]
    h2 = g_ref[...] * msgn * BN_SCALE + b_ref[...]
    h2 = jnp.maximum(h2, 0.0)
    t2_ref[:, 0:D] = h2
    pat = (lax.broadcasted_iota(jnp.int32, (ROWS, 16), 1) == 0)
    t2_ref[:, D:TW] = jnp.where(pat, 1.0, 0.0)


def _run_k2(msg0, msg1, den0, den1, expmat, gat_b, bn1_g, bn1_b):
    return pl.pallas_call(
        _k2_body,
        grid=(GRID,),
        in_specs=[
            pl.BlockSpec((ROWS, D), lambda i: (i, 0)),
            pl.BlockSpec((ROWS, D), lambda i: (i, 0)),
            pl.BlockSpec((ROWS, 16), lambda i: (i, 0)),
            pl.BlockSpec((ROWS, 16), lambda i: (i, 0)),
            pl.BlockSpec((16, D), lambda i: (0, 0)),
            pl.BlockSpec((D,), _full),
            pl.BlockSpec((D,), _full),
            pl.BlockSpec((D,), _full),
        ],
        out_specs=pl.BlockSpec((ROWS, TW), lambda i: (i, 0)),
        out_shape=jax.ShapeDtypeStruct((N, TW), jnp.float32),
    )(msg0, msg1, den0, den1, expmat, gat_b, bn1_g, bn1_b)


def _k3_body(p0_ref, p1_ref, prev_ref, wl_ref, bl_ref, wr_ref, g_ref, b_ref,
             t_ref):
    s = p0_ref[...] + p1_ref[...]
    cnt = jnp.maximum(s[:, D:D + 1], 1.0)
    agg = s[:, 0:D] / cnt
    h = jnp.dot(agg, wl_ref[...], preferred_element_type=jnp.float32)
    h = h + bl_ref[...]
    h = h + jnp.dot(prev_ref[:, 0:D], wr_ref[...],
                    preferred_element_type=jnp.float32)
    h = g_ref[...] * h * BN_SCALE + b_ref[...]
    h = jnp.maximum(h, 0.0)
    t_ref[:, 0:D] = h
    pat = (lax.broadcasted_iota(jnp.int32, (ROWS, 16), 1) == 0)
    t_ref[:, D:TW] = jnp.where(pat, 1.0, 0.0)


def _run_k3(p0, p1, prev, wl, bl, wr, g, b):
    return pl.pallas_call(
        _k3_body,
        grid=(GRID,),
        in_specs=[
            pl.BlockSpec((ROWS, TW), lambda i: (i, 0)),
            pl.BlockSpec((ROWS, TW), lambda i: (i, 0)),
            pl.BlockSpec((ROWS, TW), lambda i: (i, 0)),
            pl.BlockSpec((D, D), lambda i: (0, 0)),
            pl.BlockSpec((D,), _full),
            pl.BlockSpec((D, D), lambda i: (0, 0)),
            pl.BlockSpec((D,), _full),
            pl.BlockSpec((D,), _full),
        ],
        out_specs=pl.BlockSpec((ROWS, TW), lambda i: (i, 0)),
        out_shape=jax.ShapeDtypeStruct((N, TW), jnp.float32),
    )(p0, p1, prev, wl, bl, wr, g, b)


def _k4_body(p0_ref, p1_ref, prev_ref, wl_ref, bl_ref, wr_ref, g_ref, b_ref,
             oh_ref, c1a_ref, c1b_ref, c1bias_ref, c2w_ref, c2b_ref,
             c3w_ref, c3b_ref, out_ref, meansum, maxacc, cntacc):
    i = pl.program_id(0)

    s = p0_ref[...] + p1_ref[...]
    cnt = jnp.maximum(s[:, D:D + 1], 1.0)
    agg = s[:, 0:D] / cnt
    h = jnp.dot(agg, wl_ref[...], preferred_element_type=jnp.float32)
    h = h + bl_ref[...]
    h = h + jnp.dot(prev_ref[:, 0:D], wr_ref[...],
                    preferred_element_type=jnp.float32)
    h = g_ref[...] * h * BN_SCALE + b_ref[...]
    h = jnp.maximum(h, 0.0)

    @pl.when(i == 0)
    def _():
        meansum[...] = jnp.zeros((G, D), jnp.float32)
        maxacc[...] = jnp.full((G, D), -jnp.inf, jnp.float32)
        cntacc[...] = jnp.zeros((G, D), jnp.float32)

    oh = oh_ref[...]
    dn = (((0,), (0,)), ((), ()))
    meansum[...] += lax.dot_general(oh, h, dn,
                                    preferred_element_type=jnp.float32)
    cntacc[...] += lax.dot_general(oh, jnp.ones((ROWS, D), jnp.float32), dn,
                                   preferred_element_type=jnp.float32)
    for gidx in range(G):
        m = oh[:, gidx:gidx + 1] > 0.0
        colmax = jnp.max(jnp.where(m, h, -jnp.inf), axis=0, keepdims=True)
        maxacc[gidx:gidx + 1, :] = jnp.maximum(maxacc[gidx:gidx + 1, :],
                                               colmax)

    @pl.when(i == GRID - 1)
    def _():
        cnt = cntacc[...]
        meanp = meansum[...] / jnp.maximum(cnt, 1.0)
        maxp = jnp.where(cnt > 0.0, maxacc[...], 0.0)
        g1 = jnp.dot(meanp, c1a_ref[...], preferred_element_type=jnp.float32)
        g1 = g1 + jnp.dot(maxp, c1b_ref[...],
                          preferred_element_type=jnp.float32)
        g1 = jnp.maximum(g1 + c1bias_ref[...], 0.0)
        g2 = jnp.dot(g1, c2w_ref[...], preferred_element_type=jnp.float32)
        g2 = jnp.maximum(g2 + c2b_ref[...], 0.0)
        g3 = jnp.dot(g2, c3w_ref[...], preferred_element_type=jnp.float32)
        out_ref[...] = g3 + c3b_ref[...]


def _run_k4(p0, p1, prev, wl, bl, wr, g, b, onehot, c1a, c1b, c1_b, c2_W,
            c2_b, c3_W, c3_b):
    return pl.pallas_call(
        _k4_body,
        grid=(GRID,),
        in_specs=[
            pl.BlockSpec((ROWS, TW), lambda i: (i, 0)),
            pl.BlockSpec((ROWS, TW), lambda i: (i, 0)),
            pl.BlockSpec((ROWS, TW), lambda i: (i, 0)),
            pl.BlockSpec((D, D), lambda i: (0, 0)),
            pl.BlockSpec((D,), _full),
            pl.BlockSpec((D, D), lambda i: (0, 0)),
            pl.BlockSpec((D,), _full),
            pl.BlockSpec((D,), _full),
            pl.BlockSpec((ROWS, G), lambda i: (i, 0)),
            pl.BlockSpec((D, D), lambda i: (0, 0)),
            pl.BlockSpec((D, D), lambda i: (0, 0)),
            pl.BlockSpec((D,), _full),
            pl.BlockSpec((D, G), lambda i: (0, 0)),
            pl.BlockSpec((G,), _full),
            pl.BlockSpec((G, 2), lambda i: (0, 0)),
            pl.BlockSpec((2,), _full),
        ],
        out_specs=pl.BlockSpec((G, 2), lambda i: (0, 0)),
        out_shape=jax.ShapeDtypeStruct((G, 2), jnp.float32),
        scratch_shapes=[
            pltpu.VMEM((G, D), jnp.float32),
            pltpu.VMEM((G, D), jnp.float32),
            pltpu.VMEM((G, D), jnp.float32),
        ],
    )(p0, p1, prev, wl, bl, wr, g, b, onehot, c1a, c1b, c1_b, c2_W, c2_b,
      c3_W, c3_b)


def _gat_edge_pass_xla(tsrc, tad, src, dst):
    h1 = tsrc[:, 0:D]
    a_s = tsrc[:, D:D + H]
    a_d = tad[:, 0:H]
    alpha = a_s[src] + a_d[dst]
    alpha = jnp.where(alpha > 0, alpha, 0.2 * alpha)
    ex = jnp.exp(alpha)
    den = jax.ops.segment_sum(ex, dst, num_segments=N)
    msg = h1[src] * jnp.repeat(ex, DH, axis=1).reshape(E, D)
    msgsum = jax.ops.segment_sum(msg, dst, num_segments=N)
    den16 = jnp.pad(den, ((0, 0), (0, 12)))
    return msgsum, den16


def _sage_agg_xla(t, src, dst):
    return jax.ops.segment_sum(t[src], dst, num_segments=N)


def kernel(x, edge_index, batch, enc_W, enc_b, enc_g, enc_beta, gat_W,
           gat_asrc, gat_adst, gat_b, bn1_g, bn1_b, sage2_Wl, sage2_bl,
           sage2_Wr, bn2_g, bn2_b, sage3_Wl, sage3_bl, sage3_Wr, bn3_g,
           bn3_b, c1_W, c1_b, c2_W, c2_b, c3_W, c3_b):
    src = edge_index[0]
    dst = edge_index[1]

    rows128 = jnp.arange(D)
    heads = rows128 // DH
    abd_s = jnp.zeros((D, 16), jnp.float32).at[rows128, heads].set(
        gat_asrc.reshape(-1))
    abd_d = jnp.zeros((D, 16), jnp.float32).at[rows128, heads].set(
        gat_adst.reshape(-1))
    expmat = jnp.zeros((16, D), jnp.float32).at[heads, rows128].set(1.0)
    onehot = (batch[:, None] == jnp.arange(G)[None, :]).astype(jnp.float32)

    tsrc, tad = _run_k1(x, enc_W, enc_b, enc_g, enc_beta, gat_W, abd_s,
                        abd_d)

    msgsum, den = _gat_edge_pass_xla(tsrc, tad, src, dst)
    zmsg = jnp.zeros((N, D), jnp.float32)
    zden = jnp.zeros((N, 16), jnp.float32)
    t2 = _run_k2(msgsum, zmsg, den, zden, expmat, gat_b, bn1_g, bn1_b)

    agg2 = _sage_agg_xla(t2, src, dst)
    zagg = jnp.zeros((N, TW), jnp.float32)
    t3 = _run_k3(agg2, zagg, t2, sage2_Wl, sage2_bl, sage2_Wr, bn2_g, bn2_b)

    agg3 = _sage_agg_xla(t3, src, dst)
    out = _run_k4(agg3, zagg, t3, sage3_Wl, sage3_bl, sage3_Wr, bn3_g, bn3_b,
                  onehot, c1_W[0:D, :], c1_W[D:2 * D, :], c1_b, c2_W, c2_b,
                  c3_W, c3_b)
    return out
